# Initial kernel scaffold; baseline (speedup 1.0000x reference)
import jax
import jax.numpy as jnp
from jax.experimental import pallas as pl


def kernel(h_mf_new, h_bp_new, h_cc_new, mf_edge_index, bp_edge_index, cc_edge_index, W_mf, b_mf, W_bp, b_bp, W_cc, b_cc):
    raise NotImplementedError("write your pallas kernel here")


# speedup vs baseline: 5.5727x; 5.5727x over previous
import functools

import jax
import jax.numpy as jnp
from jax import lax
from jax.experimental import pallas as pl
from jax.experimental.pallas import tpu as pltpu
from jax.experimental.pallas import tpu_sc as plsc

N = 10000
E = 320000
D = 128
NC = 2
NS = 16
K = 80
EPS = E // (NC * NS)
CH = EPS // K
NP = 10240
RT = NP // NS
ZR = 64


def _matmul_body(h_ref, w_ref, o_ref):
    o_ref[...] = jnp.dot(h_ref[0], w_ref[0],
                         preferred_element_type=jnp.float32)[None]


def _matmul(hs, ws):
    blk = 1000
    return pl.pallas_call(
        _matmul_body,
        grid=(3, N // blk),
        in_specs=[
            pl.BlockSpec((1, blk, D), lambda g, i: (g, i, 0)),
            pl.BlockSpec((1, D, D), lambda g, i: (g, 0, 0)),
        ],
        out_specs=pl.BlockSpec((1, blk, D), lambda g, i: (g, i, 0)),
        out_shape=jax.ShapeDtypeStruct((3, N, D), jnp.float32),
    )(hs, ws)


def _finalize_body(p_ref, h_ref, b_ref, o_ref):
    agg = p_ref[0, 0] + p_ref[0, 1] + b_ref[0, 0][None, :]
    o_ref[...] = (jnp.maximum(agg, 0.0) + h_ref[0])[None]


def _finalize(parts, hs, bs):
    blk = 1000
    return pl.pallas_call(
        _finalize_body,
        grid=(3, N // blk),
        in_specs=[
            pl.BlockSpec((1, 2, blk, D), lambda g, i: (g, 0, i, 0)),
            pl.BlockSpec((1, blk, D), lambda g, i: (g, i, 0)),
            pl.BlockSpec((1, 1, D), lambda g, i: (g, 0, 0)),
        ],
        out_specs=pl.BlockSpec((1, blk, D), lambda g, i: (g, i, 0)),
        out_shape=jax.ShapeDtypeStruct((3, N, D), jnp.float32),
    )(parts, hs, bs)


def _sc_body(hw0, hw1, hw2, s0, s1, s2, d0, d1, d2, p0, p1, p2,
             acc, sidx, didx, rows, zbuf, obuf, sem):
    cid = lax.axis_index("c")
    sid = lax.axis_index("s")
    wid = cid * NS + sid

    def zlp(i, c):
        zbuf[i // (D // 16), pl.ds((i % (D // 16)) * 16, 16)] = (
            jnp.zeros((16,), jnp.float32))
        return c
    lax.fori_loop(0, ZR * (D // 16), zlp, 0)

    for hw, se, de, pe in ((hw0, s0, d0, p0),
                           (hw1, s1, d1, p1),
                           (hw2, s2, d2, p2)):
        def zero(z, c):
            pltpu.sync_copy(zbuf, acc.at[pl.ds(sid * RT + z * ZR, ZR)])
            return c
        lax.fori_loop(0, RT // ZR, zero, 0)
        plsc.subcore_barrier()

        pltpu.sync_copy(se.at[wid], sidx)
        pltpu.sync_copy(de.at[wid], didx)

        def chunk(ch, c):
            pltpu.async_copy(hw.at[sidx.at[ch]], rows, sem).wait()
            pltpu.sync_copy(rows, acc.at[didx.at[ch]], add=True)
            return c
        lax.fori_loop(0, CH, chunk, 0)
        plsc.subcore_barrier()

        def wout(z, c):
            r0 = sid * RT + z * ZR
            pltpu.sync_copy(acc.at[pl.ds(r0, ZR)], obuf)
            pltpu.sync_copy(obuf, pe.at[cid, pl.ds(r0, ZR)])
            return c
        lax.fori_loop(0, RT // ZR, wout, 0)
        plsc.subcore_barrier()


_sc_call = pl.kernel(
    _sc_body,
    out_type=[jax.ShapeDtypeStruct((NC, NP, D), jnp.float32)] * 3,
    mesh=plsc.VectorSubcoreMesh(core_axis_name="c", subcore_axis_name="s"),
    compiler_params=pltpu.CompilerParams(use_tc_tiling_on_sc=False),
    scratch_types=[
        pltpu.VMEM_SHARED((NP, D), jnp.float32),
        pltpu.VMEM((CH, K), jnp.int32),
        pltpu.VMEM((CH, K), jnp.int32),
        pltpu.VMEM((K, D), jnp.float32),
        pltpu.VMEM((ZR, D), jnp.float32),
        pltpu.VMEM((ZR, D), jnp.float32),
        pltpu.SemaphoreType.DMA,
    ],
)


def kernel(h_mf_new, h_bp_new, h_cc_new, mf_edge_index, bp_edge_index,
           cc_edge_index, W_mf, b_mf, W_bp, b_bp, W_cc, b_cc):
    hs = jnp.stack([h_mf_new, h_bp_new, h_cc_new])
    ws = jnp.stack([W_mf, W_bp, W_cc])
    bs = jnp.stack([b_mf, b_bp, b_cc]).reshape(3, 1, D)

    hw = _matmul(hs, ws)

    def _idx(ei):
        e = ei.astype(jnp.int32)
        return (e[0].reshape(NC * NS, CH, K), e[1].reshape(NC * NS, CH, K))

    s0, d0 = _idx(mf_edge_index)
    s1, d1 = _idx(bp_edge_index)
    s2, d2 = _idx(cc_edge_index)

    p0, p1, p2 = _sc_call(hw[0], hw[1], hw[2], s0, s1, s2, d0, d1, d2)

    out = _finalize(jnp.stack([p0, p1, p2]), hs, bs)
    return (out[0], out[1], out[2])
